# baseline (speedup 1.0000x reference)
import jax
import jax.numpy as jnp
from jax.experimental import pallas as pl
from jax.experimental.pallas import tpu as pltpu


def _chamfer_body(hi_ref, lo_ref, y_ref, out_ref):
    n = pl.program_id(0)
    c = 0.5 * (hi_ref[0] + lo_ref[0])
    y = y_ref[...]
    d = (c[:, :, None] - y) ** 2
    chx = jnp.mean(jnp.min(jnp.min(d, axis=2), axis=1))
    chy = jnp.mean(jnp.min(d, axis=0))
    v = (chx + chy) * 0.125

    @pl.when(n == 0)
    def _():
        out_ref[0, 0] = 0.0

    out_ref[0, 0] += v


def kernel(bins, target_depth_maps):
    N, B, _, _ = bins.shape
    b2 = bins.reshape(N, B)
    hi = b2[:, 1:].reshape(N, B - 1, 1)
    lo = b2[:, :-1].reshape(N, B - 1, 1)
    y = target_depth_maps[:, 0, 112:, 112:].reshape(N, 98, 128)

    out = pl.pallas_call(
        _chamfer_body,
        grid=(N,),
        in_specs=[
            pl.BlockSpec((1, B - 1, 1), lambda n: (n, 0, 0)),
            pl.BlockSpec((1, B - 1, 1), lambda n: (n, 0, 0)),
            pl.BlockSpec((1, 98, 128), lambda n: (n, 0, 0)),
        ],
        out_specs=pl.BlockSpec(memory_space=pltpu.SMEM),
        out_shape=jax.ShapeDtypeStruct((1, 1), jnp.float32),
    )(hi, lo, y)
    return out[0, 0]
